# Initial kernel scaffold; baseline (speedup 1.0000x reference)
import jax
import jax.numpy as jnp
from jax.experimental import pallas as pl


def kernel(x, text_embedding, params):
    raise NotImplementedError("write your pallas kernel here")


# speedup vs baseline: 3.5284x; 3.5284x over previous
import numpy as np
import jax
import jax.numpy as jnp
from jax.experimental import pallas as pl
from jax.experimental.pallas import tpu as pltpu

D = 256
H = 32
W = 32
N = 1024
MODES = 8
TEXT = 768
E = 4
TOPK = 2
HID = 1024
FNO_L = 2
KNO_L = 8
MLP_L = 8
OF_L = 6
NH = 4
HD = 64
B = 4

_PREC = jax.lax.Precision.DEFAULT
_PREC_FFT = jax.lax.Precision.HIGHEST
_IC = 32
_NC = D // _IC


def _build_fno_consts():
    ky = np.zeros(128, np.int64)
    kx = np.zeros(128, np.int64)
    for blk in range(2):
        for a in range(MODES):
            for b in range(MODES):
                m = blk * 64 + a * MODES + b
                ky[m] = a if blk == 0 else H - MODES + a
                kx[m] = b
    hh = np.arange(N) // W
    ww = np.arange(N) % W
    theta = 2 * np.pi * (ky[None, :] * hh[:, None] + kx[None, :] * ww[:, None]) / W
    fre_t = np.cos(theta)
    fim_t = -np.sin(theta)
    c = np.where(kx == 0, 1.0, 2.0)
    ang_x = 2 * np.pi * kx[None, :] * ww[:, None] / W
    c_r = c[None, :] * np.cos(ang_x)
    c_i = -c[None, :] * np.sin(ang_x)
    ang_y = 2 * np.pi * ky[None, :] * hh[:, None] / H
    cy = np.cos(ang_y)
    sy = np.sin(ang_y)
    are = (c_r * cy + c_i * sy) / N
    aim = (-c_r * sy + c_i * cy) / N
    m1 = np.zeros((N, 256))
    m2 = np.zeros((N, 256))
    m1[:, 0::2] = are
    m1[:, 1::2] = -are
    m2[:, 0::2] = aim
    m2[:, 1::2] = aim
    p = np.zeros((256, 256))
    for m in range(128):
        p[m, 2 * m] = 1.0
        p[128 + m, 2 * m + 1] = 1.0
    f32 = np.float32
    return f32(fre_t), f32(fim_t), f32(m1), f32(m2), f32(p)


_FRE_T, _FIM_T, _M1, _M2, _P = _build_fno_consts()


def _gelu(x):
    return 0.5 * x * (1.0 + jax.lax.erf(x * np.float32(2 ** -0.5)))


def _ln(x, g, b, eps=1e-5):
    m = jnp.mean(x, axis=-1, keepdims=True)
    v = jnp.mean((x - m) ** 2, axis=-1, keepdims=True)
    return (x - m) / jnp.sqrt(v + eps) * g + b


def _dot(a, b):
    return jnp.dot(a, b, precision=_PREC)


def _gate_kernel(x_ref, text_ref, g1w_ref, g1b_ref, g2w_ref, g2b_ref, dense_ref):
    for b in range(B):
        xm = jnp.mean(x_ref[b], axis=0, keepdims=True)
        feat = jnp.concatenate([xm, text_ref[b:b + 1, :]], axis=1)
        hg = jnp.maximum(_dot(feat, g1w_ref[...]) + g1b_ref[...], 0.0)
        logits = _dot(hg, g2w_ref[...]) + g2b_ref[...]
        iota = jax.lax.broadcasted_iota(jnp.int32, (1, E), 1)
        rank = jnp.zeros((1, E), jnp.int32)
        for k in range(E):
            lk = logits[:, k:k + 1]
            gt = (lk > logits) | ((lk == logits) & (k < iota))
            rank = rank + gt.astype(jnp.int32)
        sel = rank < TOPK
        mx = jnp.max(logits, axis=1, keepdims=True)
        ex = jnp.where(sel, jnp.exp(logits - mx), 0.0)
        dense_ref[b:b + 1, :] = ex / jnp.sum(ex, axis=1, keepdims=True)


def _gate(x, text, p):
    return pl.pallas_call(
        _gate_kernel,
        out_shape=jax.ShapeDtypeStruct((B, E), jnp.float32),
    )(x, text, p['g1w'], p['g1b'].reshape(1, D), p['g2w'], p['g2b'].reshape(1, E))


def _kno_kernel(dense_ref, x_ref, ng, nb, lw, lb, dw, db, pw, pb, out_ref):
    b = pl.program_id(0)
    w = dense_ref[b, 1]

    @pl.when(w == 0.0)
    def _():
        out_ref[...] = jnp.zeros_like(out_ref)

    @pl.when(w != 0.0)
    def _():
        xb = x_ref[0]
        for i in range(KNO_L):
            h = _ln(xb, ng[i], nb[i])
            h = _gelu(_dot(h, lw[i]) + lb[i])
            h = _gelu(_dot(h, dw[i]) + db[i])
            xb = xb + _dot(h, pw[i]) + pb[i]
        out_ref[0] = xb * w


def _mlp_kernel(dense_ref, x_ref, ng, nb, w1, b1, w2, b2, out_ref):
    b = pl.program_id(0)
    w = dense_ref[b, 2]

    @pl.when(w == 0.0)
    def _():
        out_ref[...] = jnp.zeros_like(out_ref)

    @pl.when(w != 0.0)
    def _():
        xb = x_ref[0]
        for i in range(MLP_L):
            h = _ln(xb, ng[i], nb[i])
            h = _gelu(_dot(h, w1[i]) + b1[i])
            xb = xb + _dot(h, w2[i]) + b2[i]
        out_ref[0] = xb * w


def _sample_grid_call(kern, x, dense, *weights):
    n_w = len(weights)
    return pl.pallas_call(
        kern,
        grid=(B,),
        in_specs=[
            pl.BlockSpec(memory_space=pltpu.SMEM),
            pl.BlockSpec((1, N, D), lambda b: (b, 0, 0)),
        ] + [pl.BlockSpec(w.shape, lambda b: (0,) * w.ndim) for w in weights],
        out_specs=pl.BlockSpec((1, N, D), lambda b: (b, 0, 0)),
        out_shape=jax.ShapeDtypeStruct((B, N, D), jnp.float32),
        compiler_params=pltpu.CompilerParams(
            dimension_semantics=("arbitrary",)),
    )(dense, x, *weights)


def _of_kernel(dense_ref, x_ref, n1g, n1b, qkvw, qkvb, pw, pb,
               lnkg, lnkb, lnvg, lnvb, n2g, n2b, m1w, m1b, m2w, m2b, out_ref):
    b = pl.program_id(0)
    w = dense_ref[b, 3]

    @pl.when(w == 0.0)
    def _():
        out_ref[...] = jnp.zeros_like(out_ref)

    @pl.when(w != 0.0)
    def _():
        xb = x_ref[0]
        for i in range(OF_L):
            xn = _ln(xb, n1g[i], n1b[i])
            qkv = _dot(xn, qkvw[i]) + qkvb[i]
            heads = []
            for hh in range(NH):
                qh = qkv[:, hh * HD:(hh + 1) * HD]
                kh = qkv[:, D + hh * HD:D + (hh + 1) * HD]
                vh = qkv[:, 2 * D + hh * HD:2 * D + (hh + 1) * HD]
                kh = _ln(kh, lnkg[i], lnkb[i])
                vh = _ln(vh, lnvg[i], lnvb[i])
                ctx = jax.lax.dot_general(kh, vh, (((0,), (0,)), ((), ())),
                                          precision=_PREC)
                heads.append(_dot(qh, ctx) * (HD ** -0.5))
            att = jnp.concatenate(heads, axis=1)
            xb = xb + _dot(att, pw[i]) + pb[i]
            h2 = _ln(xb, n2g[i], n2b[i])
            h2 = _gelu(_dot(h2, m1w[i]) + m1b[i])
            xb = xb + _dot(h2, m2w[i]) + m2b[i]
        out_ref[0] = xb * w


def _fno_kernel(dense_ref, x_ref, w1_ref, w2_ref, cw_ref, cb_ref,
                fre_ref, fim_ref, m1_ref, m2_ref, p_ref, out_ref,
                xn_s, a1_s, a2_s, t1_s, t2_s):
    l = pl.program_id(0)
    ic = pl.program_id(1)
    last = _NC - 1
    for b in range(B):
        w = dense_ref[b, 0]
        sel = w != 0.0

        @pl.when(sel & (ic == 0))
        def _start():
            @pl.when(l == 0)
            def _():
                xn_s[b] = x_ref[b]
            xb = xn_s[b]
            xre = jax.lax.dot_general(xb, fre_ref[...], (((0,), (0,)), ((), ())),
                                      precision=_PREC_FFT)
            xim = jax.lax.dot_general(xb, fim_ref[...], (((0,), (0,)), ((), ())),
                                      precision=_PREC_FFT)
            a1_s[b] = jnp.dot(jnp.concatenate([xre, xim], axis=1), p_ref[...],
                              precision=_PREC_FFT)
            a2_s[b] = jnp.dot(jnp.concatenate([xim, xre], axis=1), p_ref[...],
                              precision=_PREC_FFT)
            t1_s[b] = jnp.zeros((D, 256), jnp.float32)
            t2_s[b] = jnp.zeros((D, 256), jnp.float32)

        @pl.when(sel)
        def _acc():
            w1r = w1_ref[0]
            w2r = w2_ref[0]
            a1c = a1_s[b, pl.ds(ic * _IC, _IC), :]
            a2c = a2_s[b, pl.ds(ic * _IC, _IC), :]
            t1_s[b, :, 0:128] += jnp.sum(w1r * a1c[:, None, 0:128], axis=0)
            t2_s[b, :, 0:128] += jnp.sum(w1r * a2c[:, None, 0:128], axis=0)
            t1_s[b, :, 128:256] += jnp.sum(w2r * a1c[:, None, 128:256], axis=0)
            t2_s[b, :, 128:256] += jnp.sum(w2r * a2c[:, None, 128:256], axis=0)

        @pl.when(sel & (ic == last))
        def _finish():
            spec = (jax.lax.dot_general(m1_ref[...], t1_s[b],
                                        (((1,), (1,)), ((), ())), precision=_PREC_FFT)
                    + jax.lax.dot_general(m2_ref[...], t2_s[b],
                                          (((1,), (1,)), ((), ())), precision=_PREC_FFT))
            x2 = _dot(xn_s[b], cw_ref[l]) + cb_ref[l]
            xn_s[b] = _gelu(spec + x2)

        @pl.when((l == FNO_L - 1) & (ic == last))
        def _out():
            @pl.when(sel)
            def _():
                out_ref[b] = (xn_s[b] + x_ref[b]) * w

            @pl.when(jnp.logical_not(sel))
            def _():
                out_ref[b] = jnp.zeros((N, D), jnp.float32)


def _fno(x, dense, p):
    w1 = p['fno_w1'].reshape(FNO_L, D, D, 128)
    w2 = p['fno_w2'].reshape(FNO_L, D, D, 128)
    cb = p['fno_cb'].reshape(FNO_L, 1, D)
    f32 = jnp.float32
    return pl.pallas_call(
        _fno_kernel,
        grid=(FNO_L, _NC),
        in_specs=[
            pl.BlockSpec(memory_space=pltpu.SMEM),
            pl.BlockSpec((B, N, D), lambda l, c: (0, 0, 0)),
            pl.BlockSpec((1, _IC, D, 128), lambda l, c: (l, c, 0, 0)),
            pl.BlockSpec((1, _IC, D, 128), lambda l, c: (l, c, 0, 0)),
            pl.BlockSpec((FNO_L, D, D), lambda l, c: (0, 0, 0)),
            pl.BlockSpec((FNO_L, 1, D), lambda l, c: (0, 0, 0)),
            pl.BlockSpec((N, 128), lambda l, c: (0, 0)),
            pl.BlockSpec((N, 128), lambda l, c: (0, 0)),
            pl.BlockSpec((N, 256), lambda l, c: (0, 0)),
            pl.BlockSpec((N, 256), lambda l, c: (0, 0)),
            pl.BlockSpec((256, 256), lambda l, c: (0, 0)),
        ],
        out_specs=pl.BlockSpec((B, N, D), lambda l, c: (0, 0, 0)),
        out_shape=jax.ShapeDtypeStruct((B, N, D), f32),
        scratch_shapes=[
            pltpu.VMEM((B, N, D), f32),
            pltpu.VMEM((B, D, 256), f32),
            pltpu.VMEM((B, D, 256), f32),
            pltpu.VMEM((B, D, 256), f32),
            pltpu.VMEM((B, D, 256), f32),
        ],
        compiler_params=pltpu.CompilerParams(
            dimension_semantics=("arbitrary", "arbitrary")),
    )(dense, x, w1, w2, p['fno_cw'], cb,
      jnp.asarray(_FRE_T), jnp.asarray(_FIM_T),
      jnp.asarray(_M1), jnp.asarray(_M2), jnp.asarray(_P))


def kernel(x, text_embedding, params):
    p = params
    dense = _gate(x, text_embedding, p)
    kno = _sample_grid_call(
        _kno_kernel, x, dense,
        p['kno_ng'].reshape(KNO_L, 1, D), p['kno_nb'].reshape(KNO_L, 1, D),
        p['kno_lw'], p['kno_lb'].reshape(KNO_L, 1, 2 * D),
        p['kno_dw'], p['kno_db'].reshape(KNO_L, 1, 2 * D),
        p['kno_pw'], p['kno_pb'].reshape(KNO_L, 1, D))
    mlp = _sample_grid_call(
        _mlp_kernel, x, dense,
        p['mlp_ng'].reshape(MLP_L, 1, D), p['mlp_nb'].reshape(MLP_L, 1, D),
        p['mlp_w1'], p['mlp_b1'].reshape(MLP_L, 1, HID),
        p['mlp_w2'], p['mlp_b2'].reshape(MLP_L, 1, D))
    of = _sample_grid_call(
        _of_kernel, x, dense,
        p['of_n1g'].reshape(OF_L, 1, D), p['of_n1b'].reshape(OF_L, 1, D),
        p['of_qkvw'], p['of_qkvb'].reshape(OF_L, 1, 3 * D),
        p['of_pw'], p['of_pb'].reshape(OF_L, 1, D),
        p['of_lnkg'].reshape(OF_L, 1, HD), p['of_lnkb'].reshape(OF_L, 1, HD),
        p['of_lnvg'].reshape(OF_L, 1, HD), p['of_lnvb'].reshape(OF_L, 1, HD),
        p['of_n2g'].reshape(OF_L, 1, D), p['of_n2b'].reshape(OF_L, 1, D),
        p['of_m1w'], p['of_m1b'].reshape(OF_L, 1, HID),
        p['of_m2w'], p['of_m2b'].reshape(OF_L, 1, D))
    fno = _fno(x, dense, p)
    return fno + kno + mlp + of
